# baseline (speedup 1.0000x reference)
import functools

import jax
import jax.numpy as jnp
from jax import lax
from jax.experimental import pallas as pl
from jax.experimental.pallas import tpu as pltpu
from jax.experimental.pallas import tpu_sc as plsc

VOCAB = 1000000
ENC = 64
BATCH = 16384

NUM_CORES = 2
NUM_SUBCORES = 16
LANES = 16
NW = NUM_CORES * NUM_SUBCORES
BPW = BATCH // NW
CHUNK = 128
NCHUNK = BPW // CHUNK

_mesh = plsc.VectorSubcoreMesh(core_axis_name="c", subcore_axis_name="s")


@functools.partial(
    pl.kernel,
    mesh=_mesh,
    compiler_params=pltpu.CompilerParams(
        needs_layout_passes=False, use_tc_tiling_on_sc=False),
    out_type=jax.ShapeDtypeStruct((BATCH,), jnp.float32),
    scratch_types=[
        pltpu.VMEM((NCHUNK, CHUNK), jnp.int32),
        pltpu.VMEM((NCHUNK, CHUNK), jnp.int32),
        pltpu.VMEM((BPW, ENC), jnp.float32),
        pltpu.VMEM((BPW, ENC), jnp.float32),
        pltpu.VMEM((BPW,), jnp.float32),
        pltpu.VMEM((LANES * (LANES + 1),), jnp.float32),
        pltpu.SemaphoreType.DMA,
        pltpu.SemaphoreType.DMA,
    ],
)
def _genre2vec_sc(idx_a_hbm, idx_b_hbm, emb_hbm, ctx_hbm, out_hbm,
                  ia_v, ib_v, ra_v, rb_v, o_v, ps_v, sem_a, sem_b):
    wid = lax.axis_index("s") * NUM_CORES + lax.axis_index("c")
    base = wid * BPW

    pltpu.sync_copy(idx_a_hbm.at[wid], ia_v)
    pltpu.sync_copy(idx_b_hbm.at[wid], ib_v)

    copies = []
    for j in range(NCHUNK):
        copies.append(pltpu.async_copy(
            emb_hbm.at[ia_v.at[j]], ra_v.at[pl.ds(j * CHUNK, CHUNK)], sem_a))
        copies.append(pltpu.async_copy(
            ctx_hbm.at[ib_v.at[j]], rb_v.at[pl.ds(j * CHUNK, CHUNK)], sem_b))
    for cp in copies:
        cp.wait()

    lane_iota = lax.iota(jnp.int32, LANES)

    def group_body(g, _):
        row0 = g * LANES
        for rl in range(LANES):
            r = row0 + rl
            p = (ra_v[r, pl.ds(0, LANES)] * rb_v[r, pl.ds(0, LANES)]
                 + ra_v[r, pl.ds(LANES, LANES)] * rb_v[r, pl.ds(LANES, LANES)]
                 + ra_v[r, pl.ds(2 * LANES, LANES)] * rb_v[r, pl.ds(2 * LANES, LANES)]
                 + ra_v[r, pl.ds(3 * LANES, LANES)] * rb_v[r, pl.ds(3 * LANES, LANES)])
            ps_v[pl.ds(rl * (LANES + 1), LANES)] = p
        pitch_iota = lane_iota * (LANES + 1)
        acc = plsc.load_gather(ps_v, [pitch_iota])
        for c in range(1, LANES):
            acc = acc + plsc.load_gather(ps_v, [pitch_iota + c])
        o_v[pl.ds(row0, LANES)] = 1.0 / (1.0 + jnp.exp(-acc))
        return 0

    lax.fori_loop(0, BPW // LANES, group_body, 0)

    pltpu.sync_copy(o_v, out_hbm.at[pl.ds(base, BPW)])


def kernel(input_genres, context_genres, embedding_table, context_table):
    ia = input_genres.astype(jnp.int32).reshape(NW, NCHUNK, CHUNK)
    ib = context_genres.astype(jnp.int32).reshape(NW, NCHUNK, CHUNK)
    return _genre2vec_sc(ia, ib, embedding_table, context_table)
